# baseline (speedup 1.0000x reference)
import jax
import jax.numpy as jnp
from jax import lax
from jax.experimental import pallas as pl
from jax.experimental.pallas import tpu as pltpu
from jax.experimental.pallas import tpu_sc as plsc

HIST = 200
DP = 64
NLANE = 16
NGRP = DP // NLANE
NSUB = 16
PER_TILE = 16
NBUF = 12
NW1 = 1000000
NW2 = 100000


def _extract_col(blk_ref, col, iota16):
    colv = lax.broadcast_in_dim(col, (NLANE,), ()).astype(jnp.int32)
    return [plsc.load_gather(blk_ref, [iota16 + g * NLANE, colv])
            for g in range(NGRP)]


def _body(x0_hbm, x1_hbm, emb1t_hbm, lin_w_hbm, lin_b_hbm, emb2t_hbm,
          out_hbm,
          idx_v, ring_v, y2blk_v, x1_v, linw_v, linb_v,
          acc_v, y2acc_v, gath_v, out_v, shared,
          sem_r, sem_y, sem_w):
    sid = lax.axis_index("s")
    iota16 = lax.iota(jnp.int32, NLANE)

    @pl.when(sid == 0)
    def _():
        pltpu.async_copy(lin_w_hbm, linw_v, sem_w)
        pltpu.async_copy(lin_b_hbm, linb_v, sem_w)

    pltpu.sync_copy(x0_hbm, idx_v.at[pl.ds(0, HIST)])
    base = sid * PER_TILE
    vec = idx_v[pl.ds(base, NLANE)]

    handles = [None] * PER_TILE
    cols = [None] * PER_TILE
    acc = [jnp.zeros((NLANE,), jnp.float32) for _ in range(NGRP)]

    def _drain(j):
        handles[j].wait()
        mm = lax.broadcast_in_dim(
            jnp.where(base + j < HIST, 1.0, 0.0), (NLANE,), ())
        g_vecs = _extract_col(ring_v.at[j % NBUF], cols[j], iota16)
        for g in range(NGRP):
            acc[g] = acc[g] + g_vecs[g] * mm

    for j in range(PER_TILE):
        if j >= NBUF:
            _drain(j - NBUF)
        r = jnp.clip(vec[j], 0, NW1 - 1)
        blk = pl.multiple_of((r >> 7) << 7, 128)
        cols[j] = r - blk
        handles[j] = pltpu.async_copy(
            emb1t_hbm.at[:, pl.ds(blk, 128)], ring_v.at[j % NBUF], sem_r)
    for j in range(PER_TILE - NBUF, PER_TILE):
        _drain(j)

    for g in range(NGRP):
        acc_v[pl.ds(g * NLANE, NLANE)] = acc[g]
    pltpu.sync_copy(acc_v, shared.at[sid])

    @pl.when(sid == 1)
    def _():
        pltpu.sync_copy(x1_hbm, x1_v.at[pl.ds(0, 1)])
        r1 = jnp.clip(x1_v[pl.ds(0, NLANE)][0], 0, NW2 - 1)
        blk1 = pl.multiple_of((r1 >> 7) << 7, 128)
        pltpu.async_copy(
            emb2t_hbm.at[:, pl.ds(blk1, 128)], y2blk_v, sem_y).wait()
        y2_vecs = _extract_col(y2blk_v, r1 - blk1, iota16)
        for g in range(NGRP):
            y2acc_v[pl.ds(g * NLANE, NLANE)] = y2_vecs[g]
        pltpu.sync_copy(y2acc_v, shared.at[NSUB])

    plsc.subcore_barrier()

    @pl.when(sid == 0)
    def _():
        pltpu.sync_copy(shared, gath_v)
        s = [jnp.zeros((NLANE,), jnp.float32) for _ in range(NGRP)]
        for t in range(NSUB):
            for g in range(NGRP):
                s[g] = s[g] + gath_v[t, pl.ds(g * NLANE, NLANE)]
        y2g = [gath_v[NSUB, pl.ds(g * NLANE, NLANE)] for g in range(NGRP)]

        pltpu.make_async_copy(lin_w_hbm, linw_v, sem_w).wait()
        pltpu.make_async_copy(lin_b_hbm, linb_v, sem_w).wait()

        v = [jnp.zeros((NLANE,), jnp.float32) for _ in range(NGRP)]
        for d in range(DP):
            bd = lax.broadcast_in_dim(y2g[d // NLANE][d % NLANE],
                                      (NLANE,), ())
            for j in range(NGRP):
                v[j] = v[j] + bd * linw_v[d, pl.ds(j * NLANE, NLANE)]

        accv = jnp.zeros((NLANE,), jnp.float32)
        inv_n = 1.0 / HIST
        for j in range(NGRP):
            accv = accv + (s[j] * inv_n) * v[j]
            accv = accv + y2g[j] * linb_v[pl.ds(j * NLANE, NLANE)]

        total = accv[0]
        for i in range(1, NLANE):
            total = total + accv[i]
        tb = lax.broadcast_in_dim(total, (NLANE,), ())
        out_v[...] = 1.0 / (1.0 + jnp.exp(-tb))
        pltpu.sync_copy(out_v, out_hbm)


def kernel(x0, x1, emb1_w, lin_w, lin_b, emb2_w):
    emb1_t = emb1_w.T
    emb2_t = emb2_w.T

    mesh = plsc.VectorSubcoreMesh(
        core_axis_name="c", subcore_axis_name="s", num_cores=1)
    run = pl.kernel(
        _body,
        out_type=jax.ShapeDtypeStruct((NLANE,), jnp.float32),
        mesh=mesh,
        compiler_params=pltpu.CompilerParams(
            use_tc_tiling_on_sc=True, needs_layout_passes=False),
        scratch_types=[
            pltpu.VMEM((NSUB * PER_TILE + 8,), jnp.int32),
            pltpu.VMEM((NBUF, DP, 128), jnp.float32),
            pltpu.VMEM((DP, 128), jnp.float32),
            pltpu.VMEM((NLANE,), jnp.int32),
            pltpu.VMEM((DP, DP), jnp.float32),
            pltpu.VMEM((DP,), jnp.float32),
            pltpu.VMEM((DP,), jnp.float32),
            pltpu.VMEM((DP,), jnp.float32),
            pltpu.VMEM((NSUB + 1, DP), jnp.float32),
            pltpu.VMEM((NLANE,), jnp.float32),
            pltpu.VMEM_SHARED((NSUB + 1, DP), jnp.float32),
            pltpu.SemaphoreType.DMA,
            pltpu.SemaphoreType.DMA,
            pltpu.SemaphoreType.DMA,
        ],
    )
    out = run(x0.astype(jnp.int32), x1.astype(jnp.int32),
              emb1_t, lin_w, lin_b, emb2_t)
    return out[0]


# speedup vs baseline: 1.0599x; 1.0599x over previous
import jax
import jax.numpy as jnp
from jax import lax
from jax.experimental import pallas as pl
from jax.experimental.pallas import tpu as pltpu
from jax.experimental.pallas import tpu_sc as plsc

HIST = 200
DP = 64
NLANE = 16
NGRP = DP // NLANE
NSUB = 16
PER_TILE = 16
NBUF = 8
NW1 = 1000000
NW2 = 100000


def _extract_col(blk_ref, col, iota16):
    colv = lax.broadcast_in_dim(col, (NLANE,), ()).astype(jnp.int32)
    return [plsc.load_gather(blk_ref, [iota16 + g * NLANE, colv])
            for g in range(NGRP)]


def _body(x0_hbm, x1_hbm, emb1t_hbm, lin_w_hbm, lin_b_hbm, emb2t_hbm,
          out_hbm,
          idx_v, ring_v, y2blk_v, x1_v, linw_v, linb_v,
          acc_v, y2acc_v, gath_v, out_v, shared,
          sem_r, sem_y, sem_w):
    sid = lax.axis_index("s")
    iota16 = lax.iota(jnp.int32, NLANE)

    @pl.when(sid == 0)
    def _():
        pltpu.async_copy(lin_w_hbm, linw_v, sem_w)
        pltpu.async_copy(lin_b_hbm, linb_v, sem_w)

    @pl.when(sid == NSUB - 1)
    def _():
        pltpu.sync_copy(x1_hbm, x1_v.at[pl.ds(0, 1)])
        r1 = jnp.clip(x1_v[pl.ds(0, NLANE)][0], 0, NW2 - 1)
        blk1 = pl.multiple_of((r1 >> 7) << 7, 128)
        pltpu.async_copy(
            emb2t_hbm.at[:, pl.ds(blk1, 128)], y2blk_v, sem_y).wait()
        y2_vecs = _extract_col(y2blk_v, r1 - blk1, iota16)
        for g in range(NGRP):
            y2acc_v[pl.ds(g * NLANE, NLANE)] = y2_vecs[g]
        pltpu.sync_copy(y2acc_v, shared.at[NSUB])

    pltpu.sync_copy(x0_hbm, idx_v.at[pl.ds(0, HIST)])
    base = sid * PER_TILE
    zero16f = jnp.zeros((NLANE,), jnp.float32)
    for g in range(NGRP):
        acc_v[pl.ds(g * NLANE, NLANE)] = zero16f

    @pl.when(base < HIST)
    def _():
        vec = idx_v[pl.ds(base, NLANE)]

        handles = [None] * PER_TILE
        cols = [None] * PER_TILE
        acc = [jnp.zeros((NLANE,), jnp.float32) for _ in range(NGRP)]

        def _drain(j):
            handles[j].wait()
            mm = lax.broadcast_in_dim(
                jnp.where(base + j < HIST, 1.0, 0.0), (NLANE,), ())
            g_vecs = _extract_col(ring_v.at[j % NBUF], cols[j], iota16)
            for g in range(NGRP):
                acc[g] = acc[g] + g_vecs[g] * mm

        for j in range(PER_TILE):
            if j >= NBUF:
                _drain(j - NBUF)
            r = jnp.clip(vec[j], 0, NW1 - 1)
            blk = pl.multiple_of((r >> 7) << 7, 128)
            cols[j] = r - blk
            handles[j] = pltpu.async_copy(
                emb1t_hbm.at[:, pl.ds(blk, 128)], ring_v.at[j % NBUF],
                sem_r)
        for j in range(PER_TILE - NBUF, PER_TILE):
            _drain(j)

        for g in range(NGRP):
            acc_v[pl.ds(g * NLANE, NLANE)] = acc[g]

    pltpu.sync_copy(acc_v, shared.at[sid])

    plsc.subcore_barrier()

    @pl.when(sid == 0)
    def _():
        pltpu.sync_copy(shared, gath_v)
        s = [jnp.zeros((NLANE,), jnp.float32) for _ in range(NGRP)]
        for t in range(NSUB):
            for g in range(NGRP):
                s[g] = s[g] + gath_v[t, pl.ds(g * NLANE, NLANE)]
        y2g = [gath_v[NSUB, pl.ds(g * NLANE, NLANE)] for g in range(NGRP)]

        pltpu.make_async_copy(lin_w_hbm, linw_v, sem_w).wait()
        pltpu.make_async_copy(lin_b_hbm, linb_v, sem_w).wait()

        v = [jnp.zeros((NLANE,), jnp.float32) for _ in range(NGRP)]
        for d in range(DP):
            bd = lax.broadcast_in_dim(y2g[d // NLANE][d % NLANE],
                                      (NLANE,), ())
            for j in range(NGRP):
                v[j] = v[j] + bd * linw_v[d, pl.ds(j * NLANE, NLANE)]

        accv = jnp.zeros((NLANE,), jnp.float32)
        inv_n = 1.0 / HIST
        for j in range(NGRP):
            accv = accv + (s[j] * inv_n) * v[j]
            accv = accv + y2g[j] * linb_v[pl.ds(j * NLANE, NLANE)]

        total = accv[0]
        for i in range(1, NLANE):
            total = total + accv[i]
        tb = lax.broadcast_in_dim(total, (NLANE,), ())
        out_v[...] = 1.0 / (1.0 + jnp.exp(-tb))
        pltpu.sync_copy(out_v, out_hbm)


def kernel(x0, x1, emb1_w, lin_w, lin_b, emb2_w):
    emb1_t = emb1_w.T
    emb2_t = emb2_w.T

    mesh = plsc.VectorSubcoreMesh(
        core_axis_name="c", subcore_axis_name="s", num_cores=1)
    run = pl.kernel(
        _body,
        out_type=jax.ShapeDtypeStruct((NLANE,), jnp.float32),
        mesh=mesh,
        compiler_params=pltpu.CompilerParams(
            use_tc_tiling_on_sc=True, needs_layout_passes=False),
        scratch_types=[
            pltpu.VMEM((NSUB * PER_TILE + 8,), jnp.int32),
            pltpu.VMEM((NBUF, DP, 128), jnp.float32),
            pltpu.VMEM((DP, 128), jnp.float32),
            pltpu.VMEM((NLANE,), jnp.int32),
            pltpu.VMEM((DP, DP), jnp.float32),
            pltpu.VMEM((DP,), jnp.float32),
            pltpu.VMEM((DP,), jnp.float32),
            pltpu.VMEM((DP,), jnp.float32),
            pltpu.VMEM((NSUB + 1, DP), jnp.float32),
            pltpu.VMEM((NLANE,), jnp.float32),
            pltpu.VMEM_SHARED((NSUB + 1, DP), jnp.float32),
            pltpu.SemaphoreType.DMA,
            pltpu.SemaphoreType.DMA,
            pltpu.SemaphoreType.DMA,
        ],
    )
    out = run(x0.astype(jnp.int32), x1.astype(jnp.int32),
              emb1_t, lin_w, lin_b, emb2_t)
    return out[0]


# speedup vs baseline: 1.0796x; 1.0186x over previous
import jax
import jax.numpy as jnp
from jax import lax
from jax.experimental import pallas as pl
from jax.experimental.pallas import tpu as pltpu
from jax.experimental.pallas import tpu_sc as plsc

HIST = 200
DP = 64
NLANE = 16
NGRP = DP // NLANE
NSUB = 16
PER_TILE = 16
NBUF = 8
NW1 = 1000000
NW2 = 100000


def _extract_col(blk_ref, col, iota16):
    colv = lax.broadcast_in_dim(col, (NLANE,), ()).astype(jnp.int32)
    return [plsc.load_gather(blk_ref, [iota16 + g * NLANE, colv])
            for g in range(NGRP)]


def _body(x0_hbm, x1_hbm, emb1t_hbm, lin_w_hbm, lin_b_hbm, emb2t_hbm,
          out_hbm,
          idx_v, ring_v, y2blk_v, x1_v, linw_v, linb_v,
          acc_v, y2acc_v, gath_v, out_v, shared,
          sem_r, sem_y, sem_w):
    sid = lax.axis_index("s")
    iota16 = lax.iota(jnp.int32, NLANE)

    @pl.when(sid == NSUB - 1)
    def _():
        pltpu.async_copy(lin_w_hbm, linw_v, sem_w)
        pltpu.async_copy(lin_b_hbm, linb_v, sem_w)
        pltpu.sync_copy(x1_hbm, x1_v.at[pl.ds(0, 1)])
        r1 = jnp.clip(x1_v[pl.ds(0, NLANE)][0], 0, NW2 - 1)
        blk1 = pl.multiple_of((r1 >> 7) << 7, 128)
        pltpu.async_copy(
            emb2t_hbm.at[:, pl.ds(blk1, 128)], y2blk_v, sem_y).wait()
        y2g = _extract_col(y2blk_v, r1 - blk1, iota16)

        pltpu.make_async_copy(lin_w_hbm, linw_v, sem_w).wait()
        pltpu.make_async_copy(lin_b_hbm, linb_v, sem_w).wait()

        v = [jnp.zeros((NLANE,), jnp.float32) for _ in range(NGRP)]
        cv = jnp.zeros((NLANE,), jnp.float32)
        for j in range(NGRP):
            cv = cv + y2g[j] * linb_v[pl.ds(j * NLANE, NLANE)]
        for d in range(DP):
            bd = lax.broadcast_in_dim(y2g[d // NLANE][d % NLANE],
                                      (NLANE,), ())
            for j in range(NGRP):
                v[j] = v[j] + bd * linw_v[d, pl.ds(j * NLANE, NLANE)]
        for g in range(NGRP):
            y2acc_v[pl.ds(g * NLANE, NLANE)] = v[g]
        c = cv[0]
        for i in range(1, NLANE):
            c = c + cv[i]
        out_v[...] = lax.broadcast_in_dim(c, (NLANE,), ())

    pltpu.sync_copy(x0_hbm, idx_v.at[pl.ds(0, HIST)])
    base = sid * PER_TILE
    zero16f = jnp.zeros((NLANE,), jnp.float32)
    for g in range(NGRP):
        acc_v[pl.ds(g * NLANE, NLANE)] = zero16f

    @pl.when(base < HIST)
    def _():
        vec = idx_v[pl.ds(base, NLANE)]

        handles = [None] * PER_TILE
        cols = [None] * PER_TILE
        acc = [jnp.zeros((NLANE,), jnp.float32) for _ in range(NGRP)]

        def _drain(j):
            handles[j].wait()
            mm = lax.broadcast_in_dim(
                jnp.where(base + j < HIST, 1.0, 0.0), (NLANE,), ())
            g_vecs = _extract_col(ring_v.at[j % NBUF], cols[j], iota16)
            for g in range(NGRP):
                acc[g] = acc[g] + g_vecs[g] * mm

        for j in range(PER_TILE):
            if j >= NBUF:
                _drain(j - NBUF)
            r = jnp.clip(vec[j], 0, NW1 - 1)
            blk = pl.multiple_of((r >> 7) << 7, 128)
            cols[j] = r - blk
            handles[j] = pltpu.async_copy(
                emb1t_hbm.at[:, pl.ds(blk, 128)], ring_v.at[j % NBUF],
                sem_r)
        for j in range(PER_TILE - NBUF, PER_TILE):
            _drain(j)

        for g in range(NGRP):
            acc_v[pl.ds(g * NLANE, NLANE)] = acc[g]

    pltpu.sync_copy(acc_v, shared.at[sid])

    plsc.subcore_barrier()

    @pl.when(sid == NSUB - 1)
    def _():
        pltpu.sync_copy(shared, gath_v)
        accv = jnp.zeros((NLANE,), jnp.float32)
        inv_n = 1.0 / HIST
        for g in range(NGRP):
            s = gath_v[0, pl.ds(g * NLANE, NLANE)]
            for t in range(1, NSUB):
                s = s + gath_v[t, pl.ds(g * NLANE, NLANE)]
            accv = accv + (s * inv_n) * y2acc_v[pl.ds(g * NLANE, NLANE)]

        total = accv[0]
        for i in range(1, NLANE):
            total = total + accv[i]
        total = total + out_v[pl.ds(0, NLANE)][0]
        tb = lax.broadcast_in_dim(total, (NLANE,), ())
        out_v[...] = 1.0 / (1.0 + jnp.exp(-tb))
        pltpu.sync_copy(out_v, out_hbm)


def kernel(x0, x1, emb1_w, lin_w, lin_b, emb2_w):
    emb1_t = emb1_w.T
    emb2_t = emb2_w.T

    mesh = plsc.VectorSubcoreMesh(
        core_axis_name="c", subcore_axis_name="s", num_cores=1)
    run = pl.kernel(
        _body,
        out_type=jax.ShapeDtypeStruct((NLANE,), jnp.float32),
        mesh=mesh,
        compiler_params=pltpu.CompilerParams(
            use_tc_tiling_on_sc=True, needs_layout_passes=False),
        scratch_types=[
            pltpu.VMEM((NSUB * PER_TILE + 8,), jnp.int32),
            pltpu.VMEM((NBUF, DP, 128), jnp.float32),
            pltpu.VMEM((DP, 128), jnp.float32),
            pltpu.VMEM((NLANE,), jnp.int32),
            pltpu.VMEM((DP, DP), jnp.float32),
            pltpu.VMEM((DP,), jnp.float32),
            pltpu.VMEM((DP,), jnp.float32),
            pltpu.VMEM((DP,), jnp.float32),
            pltpu.VMEM((NSUB, DP), jnp.float32),
            pltpu.VMEM((NLANE,), jnp.float32),
            pltpu.VMEM_SHARED((NSUB, DP), jnp.float32),
            pltpu.SemaphoreType.DMA,
            pltpu.SemaphoreType.DMA,
            pltpu.SemaphoreType.DMA,
        ],
    )
    out = run(x0.astype(jnp.int32), x1.astype(jnp.int32),
              emb1_t, lin_w, lin_b, emb2_t)
    return out[0]
